# baseline (speedup 1.0000x reference)
import functools

import jax
import jax.numpy as jnp
from jax import lax
from jax.experimental import pallas as pl
from jax.experimental.pallas import tpu as pltpu
from jax.experimental.pallas import tpu_sc as plsc

N = 10000
F = 128
E = 320000
NP = 10240
RP = 2560
EP = RP * 128
NW = 32
RT = RP // NW
RPT = NP // 16

_mesh = plsc.VectorSubcoreMesh(core_axis_name="c", subcore_axis_name="s")


@functools.partial(
    pl.kernel,
    out_type=jax.ShapeDtypeStruct((2, NP), jnp.float32),
    mesh=_mesh,
    scratch_types=[
        pltpu.VMEM((RT, 128), jnp.int32),
        pltpu.VMEM((128,), jnp.float32),
        pltpu.VMEM((RPT,), jnp.float32),
        pltpu.VMEM_SHARED((NP,), jnp.float32),
    ],
)
def _deg_kernel(dstm, out, idx_v, ones_v, zer_v, deg_sm):
    c = lax.axis_index("c")
    s = lax.axis_index("s")
    w = s * 2 + c

    def fill_ones(i, carry):
        ones_v[pl.ds(i * 16, 16)] = jnp.ones((16,), jnp.float32)
        return carry

    lax.fori_loop(0, 128 // 16, fill_ones, 0)

    def fill_zeros(i, carry):
        zer_v[pl.ds(i * 16, 16)] = jnp.zeros((16,), jnp.float32)
        return carry

    lax.fori_loop(0, RPT // 16, fill_zeros, 0)
    pltpu.sync_copy(zer_v, deg_sm.at[pl.ds(s * RPT, RPT)])
    pltpu.sync_copy(dstm.at[pl.ds(w * RT, RT)], idx_v)
    plsc.subcore_barrier()

    def body(j, carry):
        pltpu.sync_copy(ones_v, deg_sm.at[idx_v.at[j]], add=True)
        return carry

    lax.fori_loop(0, RT, body, 0)
    plsc.subcore_barrier()
    pltpu.sync_copy(deg_sm.at[pl.ds(s * RPT, RPT)],
                    out.at[c, pl.ds(s * RPT, RPT)])


@functools.partial(
    pl.kernel,
    out_type=jax.ShapeDtypeStruct((2, NP, F), jnp.float32),
    mesh=_mesh,
    scratch_types=[
        pltpu.VMEM((RT // 2, 128), jnp.int32),
        pltpu.VMEM((RT // 2, 128), jnp.int32),
        pltpu.VMEM((128, F), jnp.float32),
        pltpu.VMEM((128, F), jnp.float32),
        pltpu.VMEM_SHARED((NP, F), jnp.float32),
        pltpu.SemaphoreType.DMA,
        pltpu.SemaphoreType.DMA,
        pltpu.SemaphoreType.DMA,
        pltpu.SemaphoreType.DMA,
    ],
)
def _scatter_kernel(hs, srcm, dstm, out, src_v, dst_v, rows_a, rows_b,
                    acc_sm, sem_ga, sem_gb, sem_sa, sem_sb):
    c = lax.axis_index("c")
    s = lax.axis_index("s")
    w = s * 2 + c
    rtp = RT // 2

    def zero_rows(i, carry):
        rows_a[i // 8, pl.ds((i % 8) * 16, 16)] = jnp.zeros((16,), jnp.float32)
        return carry

    lax.fori_loop(0, 128 * (F // 16), zero_rows, 0)

    def zero_acc(k, carry):
        pltpu.sync_copy(rows_a, acc_sm.at[pl.ds(s * RPT + k * 128, 128)])
        return carry

    lax.fori_loop(0, RPT // 128, zero_acc, 0)
    plsc.subcore_barrier()

    for ph in range(2):
        pltpu.sync_copy(srcm.at[pl.ds(w * RT + ph * rtp, rtp)], src_v)
        pltpu.sync_copy(dstm.at[pl.ds(w * RT + ph * rtp, rtp)], dst_v)
        pltpu.async_copy(hs.at[src_v.at[0]], rows_a, sem_ga)
        pltpu.async_copy(hs.at[src_v.at[1]], rows_b, sem_gb)
        npair = rtp // 2

        def body(i, carry):
            ja = 2 * i
            pltpu.make_async_copy(hs.at[src_v.at[ja]], rows_a, sem_ga).wait()
            pltpu.async_copy(rows_a, acc_sm.at[dst_v.at[ja]], sem_sa, add=True)
            pltpu.make_async_copy(
                hs.at[src_v.at[ja + 1]], rows_b, sem_gb).wait()
            pltpu.async_copy(
                rows_b, acc_sm.at[dst_v.at[ja + 1]], sem_sb, add=True)

            @pl.when(i < npair - 1)
            def _():
                pltpu.make_async_copy(
                    rows_a, acc_sm.at[dst_v.at[ja]], sem_sa).wait()
                pltpu.async_copy(hs.at[src_v.at[ja + 2]], rows_a, sem_ga)
                pltpu.make_async_copy(
                    rows_b, acc_sm.at[dst_v.at[ja + 1]], sem_sb).wait()
                pltpu.async_copy(hs.at[src_v.at[ja + 3]], rows_b, sem_gb)

            return carry

        lax.fori_loop(0, npair, body, 0)
        pltpu.make_async_copy(
            rows_a, acc_sm.at[dst_v.at[rtp - 2]], sem_sa).wait()
        pltpu.make_async_copy(
            rows_b, acc_sm.at[dst_v.at[rtp - 1]], sem_sb).wait()
    plsc.subcore_barrier()

    def write_out(k, carry):
        pltpu.sync_copy(acc_sm.at[pl.ds(s * RPT + k * 128, 128)],
                        out.at[c, pl.ds(s * RPT + k * 128, 128)])
        return carry

    lax.fori_loop(0, RPT // 128, write_out, 0)


def _dinv_col(deg_ref):
    deg = deg_ref[0] + deg_ref[1] + 1.0
    return lax.rsqrt(deg)


def _tc_a1_body(x_ref, w1_ref, h_ref):
    h_ref[...] = jnp.dot(x_ref[...], w1_ref[...],
                         preferred_element_type=jnp.float32)


def _tc_a2_body(h_ref, deg_ref, hs_ref):
    hs_ref[...] = h_ref[...] * _dinv_col(deg_ref)


def _tc_b_body(acc_ref, hs_ref, deg_ref, w2_ref, b1_ref, hs2_ref):
    dinv = _dinv_col(deg_ref)
    z = jax.nn.relu(dinv * (acc_ref[0] + acc_ref[1] + hs_ref[...]) + b1_ref[...])
    h2 = jnp.dot(z, w2_ref[...], preferred_element_type=jnp.float32)
    hs2_ref[...] = h2 * dinv


def _tc_c_body(acc_ref, hs2_ref, deg_ref, b2_ref, out_ref):
    dinv = _dinv_col(deg_ref)
    out_ref[...] = jax.nn.relu(
        dinv * (acc_ref[0] + acc_ref[1] + hs2_ref[...]) + b2_ref[...])


_tc_a1 = pl.pallas_call(
    _tc_a1_body, out_shape=jax.ShapeDtypeStruct((NP, F), jnp.float32))
_tc_a2 = pl.pallas_call(
    _tc_a2_body, out_shape=jax.ShapeDtypeStruct((NP, F), jnp.float32))
_tc_b = pl.pallas_call(
    _tc_b_body, out_shape=jax.ShapeDtypeStruct((NP, F), jnp.float32))
_tc_c = pl.pallas_call(
    _tc_c_body, out_shape=jax.ShapeDtypeStruct((NP, F), jnp.float32))


@jax.jit
def _impl(x, edge_index, W1, b1, W2, b2):
    src = edge_index[0]
    dst = edge_index[1]
    pad = EP - E
    ar = jnp.arange(pad, dtype=jnp.int32)
    srcm = jnp.concatenate([src, ar % N]).reshape(RP, 128)
    dstm = jnp.concatenate([dst, N + ar % (NP - N)]).reshape(RP, 128)
    xp = jnp.zeros((NP, F), jnp.float32).at[:N].set(x)

    deg = _deg_kernel(dstm).reshape(2, NP, 1)
    h1 = _tc_a1(xp, W1)
    hs1 = _tc_a2(h1, deg)
    acc1 = _scatter_kernel(hs1, srcm, dstm)
    hs2 = _tc_b(acc1, hs1, deg, W2, b1)
    acc2 = _scatter_kernel(hs2, srcm, dstm)
    out = _tc_c(acc2, hs2, deg, b2)
    return out[:N]


def kernel(x, edge_index, W1, b1, W2, b2):
    return _impl(x, edge_index, W1, b1, W2, b2)


# speedup vs baseline: 1.0032x; 1.0032x over previous
import functools

import jax
import jax.numpy as jnp
from jax import lax
from jax.experimental import pallas as pl
from jax.experimental.pallas import tpu as pltpu
from jax.experimental.pallas import tpu_sc as plsc

N = 10000
F = 128
E = 320000
NP = 10240
RP = 2560
EP = RP * 128
NW = 32
RT = RP // NW
RPT = NP // 16

_mesh = plsc.VectorSubcoreMesh(core_axis_name="c", subcore_axis_name="s")


@functools.partial(
    pl.kernel,
    out_type=jax.ShapeDtypeStruct((2, NP), jnp.float32),
    mesh=_mesh,
    scratch_types=[
        pltpu.VMEM((RT, 128), jnp.int32),
        pltpu.VMEM((128,), jnp.float32),
        pltpu.VMEM((RPT,), jnp.float32),
        pltpu.VMEM_SHARED((NP,), jnp.float32),
    ],
)
def _deg_kernel(dstm, out, idx_v, ones_v, zer_v, deg_sm):
    c = lax.axis_index("c")
    s = lax.axis_index("s")
    w = s * 2 + c

    def fill_ones(i, carry):
        ones_v[pl.ds(i * 16, 16)] = jnp.ones((16,), jnp.float32)
        return carry

    lax.fori_loop(0, 128 // 16, fill_ones, 0)

    def fill_zeros(i, carry):
        zer_v[pl.ds(i * 16, 16)] = jnp.zeros((16,), jnp.float32)
        return carry

    lax.fori_loop(0, RPT // 16, fill_zeros, 0)
    pltpu.sync_copy(zer_v, deg_sm.at[pl.ds(s * RPT, RPT)])
    pltpu.sync_copy(dstm.at[pl.ds(w * RT, RT)], idx_v)
    plsc.subcore_barrier()

    def body(j, carry):
        pltpu.sync_copy(ones_v, deg_sm.at[idx_v.at[j]], add=True)
        return carry

    lax.fori_loop(0, RT, body, 0)
    plsc.subcore_barrier()
    pltpu.sync_copy(deg_sm.at[pl.ds(s * RPT, RPT)],
                    out.at[c, pl.ds(s * RPT, RPT)])


@functools.partial(
    pl.kernel,
    out_type=jax.ShapeDtypeStruct((2, NP, F), jnp.float32),
    mesh=_mesh,
    scratch_types=[
        pltpu.VMEM((RT // 2, 128), jnp.int32),
        pltpu.VMEM((RT // 2, 128), jnp.int32),
        pltpu.VMEM((128, F), jnp.float32),
        pltpu.VMEM((128, F), jnp.float32),
        pltpu.VMEM_SHARED((NP, F), jnp.float32),
        pltpu.SemaphoreType.DMA,
        pltpu.SemaphoreType.DMA,
        pltpu.SemaphoreType.DMA,
        pltpu.SemaphoreType.DMA,
    ],
)
def _scatter_kernel(hs, srcm, dstm, out, src_v, dst_v, rows_a, rows_b,
                    acc_sm, sem_ga, sem_gb, sem_sa, sem_sb):
    c = lax.axis_index("c")
    s = lax.axis_index("s")
    w = s * 2 + c
    rtp = RT // 2

    def zero_rows(i, carry):
        rows_a[i // 8, pl.ds((i % 8) * 16, 16)] = jnp.zeros((16,), jnp.float32)
        return carry

    lax.fori_loop(0, 128 * (F // 16), zero_rows, 0)

    def zero_acc(k, carry):
        pltpu.sync_copy(rows_a, acc_sm.at[pl.ds(s * RPT + k * 128, 128)])
        return carry

    lax.fori_loop(0, RPT // 128, zero_acc, 0)
    plsc.subcore_barrier()

    for ph in range(2):
        pltpu.sync_copy(srcm.at[pl.ds(w * RT + ph * rtp, rtp)], src_v)
        pltpu.sync_copy(dstm.at[pl.ds(w * RT + ph * rtp, rtp)], dst_v)
        pltpu.async_copy(hs.at[src_v.at[0]], rows_a, sem_ga)
        pltpu.async_copy(hs.at[src_v.at[1]], rows_b, sem_gb)
        npair = rtp // 2

        def body(i, carry):
            ja = 2 * i
            pltpu.make_async_copy(hs.at[src_v.at[ja]], rows_a, sem_ga).wait()
            pltpu.async_copy(rows_a, acc_sm.at[dst_v.at[ja]], sem_sa, add=True)
            pltpu.make_async_copy(
                hs.at[src_v.at[ja + 1]], rows_b, sem_gb).wait()
            pltpu.async_copy(
                rows_b, acc_sm.at[dst_v.at[ja + 1]], sem_sb, add=True)

            @pl.when(i < npair - 1)
            def _():
                pltpu.make_async_copy(
                    rows_a, acc_sm.at[dst_v.at[ja]], sem_sa).wait()
                pltpu.async_copy(hs.at[src_v.at[ja + 2]], rows_a, sem_ga)
                pltpu.make_async_copy(
                    rows_b, acc_sm.at[dst_v.at[ja + 1]], sem_sb).wait()
                pltpu.async_copy(hs.at[src_v.at[ja + 3]], rows_b, sem_gb)

            return carry

        lax.fori_loop(0, npair, body, 0)
        pltpu.make_async_copy(
            rows_a, acc_sm.at[dst_v.at[rtp - 2]], sem_sa).wait()
        pltpu.make_async_copy(
            rows_b, acc_sm.at[dst_v.at[rtp - 1]], sem_sb).wait()
    plsc.subcore_barrier()

    def write_out(k, carry):
        pltpu.sync_copy(acc_sm.at[pl.ds(s * RPT + k * 128, 128)],
                        out.at[c, pl.ds(s * RPT + k * 128, 128)])
        return carry

    lax.fori_loop(0, RPT // 128, write_out, 0)


def _dinv_col(deg_ref):
    deg = deg_ref[0] + deg_ref[1] + 1.0
    return lax.rsqrt(deg)


def _tc_a1_body(x_ref, w1_ref, h_ref):
    h_ref[...] = jnp.dot(x_ref[...], w1_ref[...],
                         preferred_element_type=jnp.float32)


def _tc_a2_body(h_ref, deg_ref, hs_ref, dinv_ref):
    dinv = _dinv_col(deg_ref)
    dinv_ref[...] = dinv
    hs_ref[...] = h_ref[...] * dinv


def _tc_b_body(acc_ref, hs_ref, dinv_ref, w2_ref, b1_ref, hs2_ref):
    dinv = dinv_ref[...]
    z = jax.nn.relu(dinv * (acc_ref[0] + acc_ref[1] + hs_ref[...]) + b1_ref[...])
    h2 = jnp.dot(z, w2_ref[...], preferred_element_type=jnp.float32)
    hs2_ref[...] = h2 * dinv


def _tc_c_body(acc_ref, hs2_ref, dinv_ref, b2_ref, out_ref):
    out_ref[...] = jax.nn.relu(
        dinv_ref[...] * (acc_ref[0] + acc_ref[1] + hs2_ref[...]) + b2_ref[...])


_tc_a1 = pl.pallas_call(
    _tc_a1_body, out_shape=jax.ShapeDtypeStruct((NP, F), jnp.float32))
_tc_a2 = pl.pallas_call(
    _tc_a2_body, out_shape=(jax.ShapeDtypeStruct((NP, F), jnp.float32),
                            jax.ShapeDtypeStruct((NP, 1), jnp.float32)))
_tc_b = pl.pallas_call(
    _tc_b_body, out_shape=jax.ShapeDtypeStruct((NP, F), jnp.float32))
_tc_c = pl.pallas_call(
    _tc_c_body, out_shape=jax.ShapeDtypeStruct((NP, F), jnp.float32))


@jax.jit
def _impl(x, edge_index, W1, b1, W2, b2):
    src = edge_index[0]
    dst = edge_index[1]
    pad = EP - E
    ar = jnp.arange(pad, dtype=jnp.int32)
    srcm = jnp.concatenate([src, ar % N]).reshape(RP, 128)
    dstm = jnp.concatenate([dst, N + ar % (NP - N)]).reshape(RP, 128)
    xp = jnp.zeros((NP, F), jnp.float32).at[:N].set(x)

    deg = _deg_kernel(dstm).reshape(2, NP, 1)
    h1 = _tc_a1(xp, W1)
    hs1, dinv = _tc_a2(h1, deg)
    acc1 = _scatter_kernel(hs1, srcm, dstm)
    hs2 = _tc_b(acc1, hs1, dinv, W2, b1)
    acc2 = _scatter_kernel(hs2, srcm, dstm)
    out = _tc_c(acc2, hs2, dinv, b2)
    return out[:N]


def kernel(x, edge_index, W1, b1, W2, b2):
    return _impl(x, edge_index, W1, b1, W2, b2)
